# baseline (device time: 59305 ns/iter reference)
import jax
import jax.numpy as jnp
from jax import lax
from jax.experimental import pallas as pl
from jax.experimental.pallas import tpu as pltpu

N_DEV = 4


def kernel(x, dy):
    k_per, m = x.shape
    k_per2, n = dy.shape
    assert k_per == k_per2
    m_out = m // N_DEV
    nh = n // 2

    def body(x_ref, dy_ref, out_ref, dy_bf, stage, comm, send_sems, recv_sems):
        my = lax.axis_index("i")
        left = (my + N_DEV - 1) % N_DEV
        right = (my + 1) % N_DEV

        def pchunk(c, half, dy_src):
            xs = x_ref[:, pl.ds(c * m_out, m_out)].astype(jnp.bfloat16)
            dys = dy_src[:, half * nh:(half + 1) * nh]
            if dys.dtype != jnp.bfloat16:
                dys = dys.astype(jnp.bfloat16)
            return lax.dot_general(
                xs, dys,
                dimension_numbers=(((0,), (0,)), ((), ())),
                preferred_element_type=jnp.float32,
            )

        stage[0, :, :] = pchunk(
            (my + N_DEV - 1) % N_DEV, 0, dy_ref[...]).astype(jnp.bfloat16)
        stage[1, :, :] = pchunk(
            (my + 1) % N_DEV, 1, dy_ref[...]).astype(jnp.bfloat16)

        barrier_sem = pltpu.get_barrier_semaphore()
        for nbr in (left, right):
            pl.semaphore_signal(
                barrier_sem, inc=1,
                device_id=(nbr,), device_id_type=pl.DeviceIdType.MESH,
            )
        pl.semaphore_wait(barrier_sem, 2)

        def make_rdma(direction, s, src):
            return pltpu.make_async_remote_copy(
                src_ref=src,
                dst_ref=comm.at[direction, s],
                send_sem=send_sems.at[direction, s],
                recv_sem=recv_sems.at[direction, s],
                device_id=(right if direction == 0 else left,),
                device_id_type=pl.DeviceIdType.MESH,
            )

        rdma_r = make_rdma(0, 0, stage.at[0])
        rdma_l = make_rdma(1, 0, stage.at[1])
        rdma_r.start()
        rdma_l.start()

        dy_bf[...] = dy_ref[...].astype(jnp.bfloat16)

        for s in range(N_DEV - 1):
            c_r = (my + 2 * N_DEV - 2 - s) % N_DEV
            c_l = (my + 2 + s) % N_DEV
            p_r = pchunk(c_r, 0, dy_bf[...])
            p_l = pchunk(c_l, 1, dy_bf[...])

            rdma_r.wait()
            if s < N_DEV - 2:
                comm[0, s, :, :] = (
                    comm[0, s, :, :].astype(jnp.float32) + p_r
                ).astype(jnp.bfloat16)
                rdma_r = make_rdma(0, s + 1, comm.at[0, s])
                rdma_r.start()
            else:
                out_ref[:, :nh] = comm[0, s, :, :].astype(jnp.float32) + p_r

            rdma_l.wait()
            if s < N_DEV - 2:
                comm[1, s, :, :] = (
                    comm[1, s, :, :].astype(jnp.float32) + p_l
                ).astype(jnp.bfloat16)
                rdma_l = make_rdma(1, s + 1, comm.at[1, s])
                rdma_l.start()
            else:
                out_ref[:, nh:] = comm[1, s, :, :].astype(jnp.float32) + p_l

    return pl.pallas_call(
        body,
        out_shape=jax.ShapeDtypeStruct((m_out, n), jnp.float32),
        in_specs=[
            pl.BlockSpec(memory_space=pltpu.VMEM),
            pl.BlockSpec(memory_space=pltpu.VMEM),
        ],
        out_specs=pl.BlockSpec(memory_space=pltpu.VMEM),
        scratch_shapes=[
            pltpu.VMEM((k_per, n), jnp.bfloat16),
            pltpu.VMEM((2, m_out, nh), jnp.bfloat16),
            pltpu.VMEM((2, N_DEV - 1, m_out, nh), jnp.bfloat16),
            pltpu.SemaphoreType.DMA((2, N_DEV - 1)),
            pltpu.SemaphoreType.DMA((2, N_DEV - 1)),
        ],
        compiler_params=pltpu.CompilerParams(
            collective_id=0,
            vmem_limit_bytes=100 * 1024 * 1024,
        ),
    )(x, dy)


# device time: 52776 ns/iter; 1.1237x vs baseline; 1.1237x over previous
import jax
import jax.numpy as jnp
from jax import lax
from jax.experimental import pallas as pl
from jax.experimental.pallas import tpu as pltpu

N_DEV = 4
N_RING = 4


def kernel(x, dy):
    k_per, m = x.shape
    k_per2, n = dy.shape
    assert k_per == k_per2
    m_out = m // N_DEV
    nq = n // N_RING

    def body(x_ref, dy_ref, out_ref, dy_bf, stage, comm,
             send_sems, recv_sems):
        my = lax.axis_index("i")
        left = (my + N_DEV - 1) % N_DEV
        right = (my + 1) % N_DEV

        def pchunk(c, ring, dy_src):
            xs = x_ref[:, pl.ds(c * m_out, m_out)].astype(jnp.bfloat16)
            dys = dy_src[:, ring * nq:(ring + 1) * nq]
            if dys.dtype != jnp.bfloat16:
                dys = dys.astype(jnp.bfloat16)
            return lax.dot_general(
                xs, dys,
                dimension_numbers=(((0,), (0,)), ((), ())),
                preferred_element_type=jnp.float32,
            )

        def c_send0(ring):
            if ring < 2:
                return (my + N_DEV - 1) % N_DEV
            return (my + 1) % N_DEV

        def c_recv(ring, s):
            if ring < 2:
                return (my + 2 * N_DEV - 2 - s) % N_DEV
            return (my + 2 + s) % N_DEV

        def make_rdma(ring, s, src):
            return pltpu.make_async_remote_copy(
                src_ref=src,
                dst_ref=comm.at[ring, s],
                send_sem=send_sems.at[ring, s],
                recv_sem=recv_sems.at[ring, s],
                device_id=(right if ring < 2 else left,),
                device_id_type=pl.DeviceIdType.MESH,
            )

        stage[0, :, :] = pchunk(c_send0(0), 0, dy_ref[...]).astype(jnp.bfloat16)
        stage[2, :, :] = pchunk(c_send0(2), 2, dy_ref[...]).astype(jnp.bfloat16)

        barrier_sem = pltpu.get_barrier_semaphore()
        for nbr in (left, right):
            pl.semaphore_signal(
                barrier_sem, inc=1,
                device_id=(nbr,), device_id_type=pl.DeviceIdType.MESH,
            )
        pl.semaphore_wait(barrier_sem, 2)

        rdmas = [None] * N_RING
        for ring in (0, 2):
            rdmas[ring] = make_rdma(ring, 0, stage.at[ring])
            rdmas[ring].start()

        for ring in (1, 3):
            stage[ring, :, :] = pchunk(
                c_send0(ring), ring, dy_ref[...]).astype(jnp.bfloat16)
            rdmas[ring] = make_rdma(ring, 0, stage.at[ring])
            rdmas[ring].start()

        dy_bf[...] = dy_ref[...].astype(jnp.bfloat16)

        for s in range(N_DEV - 1):
            p = [pchunk(c_recv(ring, s), ring, dy_bf[...])
                 for ring in range(N_RING)]
            for ring in (0, 2, 1, 3):
                rdmas[ring].wait()
                if s < N_DEV - 2:
                    comm[ring, s, :, :] = (
                        comm[ring, s, :, :].astype(jnp.float32) + p[ring]
                    ).astype(jnp.bfloat16)
                    rdmas[ring] = make_rdma(ring, s + 1, comm.at[ring, s])
                    rdmas[ring].start()
                else:
                    out_ref[:, ring * nq:(ring + 1) * nq] = (
                        comm[ring, s, :, :].astype(jnp.float32) + p[ring]
                    )

    return pl.pallas_call(
        body,
        out_shape=jax.ShapeDtypeStruct((m_out, n), jnp.float32),
        in_specs=[
            pl.BlockSpec(memory_space=pltpu.VMEM),
            pl.BlockSpec(memory_space=pltpu.VMEM),
        ],
        out_specs=pl.BlockSpec(memory_space=pltpu.VMEM),
        scratch_shapes=[
            pltpu.VMEM((k_per, n), jnp.bfloat16),
            pltpu.VMEM((N_RING, m_out, nq), jnp.bfloat16),
            pltpu.VMEM((N_RING, N_DEV - 1, m_out, nq), jnp.bfloat16),
            pltpu.SemaphoreType.DMA((N_RING, N_DEV - 1)),
            pltpu.SemaphoreType.DMA((N_RING, N_DEV - 1)),
        ],
        compiler_params=pltpu.CompilerParams(
            collective_id=0,
            vmem_limit_bytes=100 * 1024 * 1024,
        ),
    )(x, dy)
